# Initial kernel scaffold; baseline (speedup 1.0000x reference)
import jax
import jax.numpy as jnp
from jax.experimental import pallas as pl


def kernel(x, edge_index, edge_attr, W1, root1, b1, W2, root2, b2):
    raise NotImplementedError("write your pallas kernel here")


# speedup vs baseline: 3.1405x; 3.1405x over previous
import functools

import jax
import jax.numpy as jnp
from jax import lax
from jax.experimental import pallas as pl
from jax.experimental.pallas import tpu as pltpu
from jax.experimental.pallas import tpu_sc as plsc

N_NODES = 10000
N_EDGES = 320000
D = 128
K_B = 9

NC = 2
NS = 16
NW = NC * NS

CHUNK = 128
CPW = 79
E_PAD = NW * CPW * CHUNK
N_PAD = 10240

_SC_MESH = plsc.VectorSubcoreMesh(
    core_axis_name="c", subcore_axis_name="s", num_cores=NC, num_subcores=NS)


@functools.partial(
    pl.kernel,
    out_type=jax.ShapeDtypeStruct((E_PAD, D), jnp.float32),
    mesh=_SC_MESH,
    scratch_types=[
        pltpu.VMEM((CHUNK,), jnp.int32),
        pltpu.VMEM((CHUNK, D), jnp.float32),
        pltpu.SemaphoreType.DMA,
    ],
)
def _sc_gather(x_hbm, src_hbm, out_hbm, idx_v, rows_v, sem):
    c = lax.axis_index("c")
    s = lax.axis_index("s")
    base_chunk = (c * NS + s) * CPW

    def body(i, _):
        b = pl.multiple_of((base_chunk + i) * CHUNK, CHUNK)
        pltpu.sync_copy(src_hbm.at[pl.ds(b, CHUNK)], idx_v)
        pltpu.async_copy(x_hbm.at[idx_v], rows_v, sem).wait()
        pltpu.sync_copy(rows_v, out_hbm.at[pl.ds(b, CHUNK)])
        return 0

    lax.fori_loop(0, CPW, body, 0)


@functools.partial(
    pl.kernel,
    out_type=jax.ShapeDtypeStruct((NC, N_PAD, D), jnp.float32),
    mesh=_SC_MESH,
    scratch_types=[
        pltpu.VMEM((CHUNK,), jnp.int32),
        pltpu.VMEM((CHUNK, D), jnp.float32),
        pltpu.VMEM_SHARED((N_PAD, D), jnp.float32),
    ],
)
def _sc_scatter(acc_hbm, dst_hbm, zeros_hbm, out_hbm, idx_v, buf_v, agg_sh):
    c = lax.axis_index("c")
    s = lax.axis_index("s")

    def zbody(j, _):
        r = pl.multiple_of(s * 640 + j * CHUNK, CHUNK)
        pltpu.sync_copy(zeros_hbm.at[pl.ds(r, CHUNK)], agg_sh.at[pl.ds(r, CHUNK)])
        return 0

    lax.fori_loop(0, 5, zbody, 0)
    plsc.subcore_barrier()

    base_chunk = (c * NS + s) * CPW

    def body(i, _):
        b = pl.multiple_of((base_chunk + i) * CHUNK, CHUNK)
        pltpu.sync_copy(dst_hbm.at[pl.ds(b, CHUNK)], idx_v)
        pltpu.sync_copy(acc_hbm.at[pl.ds(b, CHUNK)], buf_v)
        pltpu.sync_copy(buf_v, agg_sh.at[idx_v], add=True)
        return 0

    lax.fori_loop(0, CPW, body, 0)
    plsc.subcore_barrier()

    def wbody(j, _):
        r = pl.multiple_of(s * 640 + j * CHUNK, CHUNK)
        pltpu.sync_copy(agg_sh.at[pl.ds(r, CHUNK)], out_hbm.at[c, pl.ds(r, CHUNK)])
        return 0

    lax.fori_loop(0, 5, wbody, 0)


BE = 2048


def _mm_body(ea_ref, xg_ref, w_ref, o_ref):
    u0 = ea_ref[:, 0:1]
    u1 = ea_ref[:, 1:2]
    bx = (0.5 * (1.0 - u0) ** 2, -u0 * u0 + u0 + 0.5, 0.5 * u0 * u0)
    by = (0.5 * (1.0 - u1) ** 2, -u1 * u1 + u1 + 0.5, 0.5 * u1 * u1)
    xgb = xg_ref[...]
    acc = jnp.zeros((BE, D), jnp.float32)
    for i in range(3):
        for j in range(3):
            t = jnp.dot(xgb, w_ref[i * 3 + j], preferred_element_type=jnp.float32)
            acc = acc + (bx[i] * by[j]) * t
    rows = pl.program_id(0) * BE + lax.broadcasted_iota(jnp.int32, (BE, 1), 0)
    o_ref[...] = jnp.where(rows < N_EDGES, acc, 0.0)


def _tc_edge_mm(ea, xg, w):
    return pl.pallas_call(
        _mm_body,
        grid=(E_PAD // BE,),
        in_specs=[
            pl.BlockSpec((BE, 2), lambda i: (i, 0)),
            pl.BlockSpec((BE, D), lambda i: (i, 0)),
            pl.BlockSpec((K_B, D, D), lambda i: (0, 0, 0)),
        ],
        out_specs=pl.BlockSpec((BE, D), lambda i: (i, 0)),
        out_shape=jax.ShapeDtypeStruct((E_PAD, D), jnp.float32),
    )(ea, xg, w)


BN = 1000


def _finish_body(p_ref, x_ref, root_ref, b_ref, o_ref):
    t = jnp.dot(x_ref[...], root_ref[...], preferred_element_type=jnp.float32)
    t = t + p_ref[0] + p_ref[1] + b_ref[...]
    o_ref[...] = jnp.maximum(t, 0.0)


def _tc_finish(parts, x, root, b):
    return pl.pallas_call(
        _finish_body,
        grid=(N_NODES // BN,),
        in_specs=[
            pl.BlockSpec((NC, BN, D), lambda i: (0, i, 0)),
            pl.BlockSpec((BN, D), lambda i: (i, 0)),
            pl.BlockSpec((D, D), lambda i: (0, 0)),
            pl.BlockSpec((1, D), lambda i: (0, 0)),
        ],
        out_specs=pl.BlockSpec((BN, D), lambda i: (i, 0)),
        out_shape=jax.ShapeDtypeStruct((N_NODES, D), jnp.float32),
    )(parts, x, root, b)


def kernel(x, edge_index, edge_attr, W1, root1, b1, W2, root2, b2):
    src = edge_index[0].astype(jnp.int32)
    dst = edge_index[1].astype(jnp.int32)
    pad_e = E_PAD - N_EDGES
    src_p = jnp.pad(src, (0, pad_e))
    dst_p = jnp.pad(dst, (0, pad_e))
    ea_p = jnp.pad(edge_attr.astype(jnp.float32), ((0, pad_e), (0, 0)))
    zeros = jnp.zeros((N_PAD, D), jnp.float32)

    h = x
    for w, root, b in ((W1, root1, b1), (W2, root2, b2)):
        xg = _sc_gather(h, src_p)
        acc = _tc_edge_mm(ea_p, xg, w)
        parts = _sc_scatter(acc, dst_p, zeros)
        h = _tc_finish(parts, h, root, b.reshape(1, D))
    return h
